# baseline (speedup 1.0000x reference)
import functools
import math

import jax
import jax.numpy as jnp
from jax import lax
from jax.experimental import pallas as pl
from jax.experimental.pallas import tpu as pltpu
from jax.experimental.pallas import tpu_sc as plsc

D_MODEL = 64
SCALE = math.sqrt(D_MODEL)
LANES = 16
NB = 4
NW = 32


def _scale_block(src, dst, w):
    @pl.loop(0, w, step=2)
    def _(r0):
        for dr in range(2):
            for c in range(D_MODEL // LANES):
                s = (pl.ds(r0 + dr, 1), pl.ds(c * LANES, LANES))
                dst.at[s][...] = src.at[s][...] * SCALE


def kernel(x, table):
    b, s = x.shape
    rows_per_tile = b // NW
    rounds = rows_per_tile // NB
    mesh = plsc.VectorSubcoreMesh(core_axis_name="core", subcore_axis_name="subcore")

    @functools.partial(
        pl.kernel,
        out_type=jax.ShapeDtypeStruct((b, s, D_MODEL), table.dtype),
        mesh=mesh,
        compiler_params=pltpu.CompilerParams(use_tc_tiling_on_sc=False),
        scratch_types=(
            [pltpu.VMEM((rows_per_tile, s), jnp.int32)]
            + [pltpu.VMEM((s, D_MODEL), jnp.float32) for _ in range(2 * NB)]
            + [pltpu.SemaphoreType.DMA for _ in range(2 * NB)]
        ),
    )
    def run(table_hbm, x_hbm, out_hbm, idx_v, *bufs_and_sems):
        ibuf = bufs_and_sems[:NB]
        obuf = bufs_and_sems[NB:2 * NB]
        gsem = bufs_and_sems[2 * NB:3 * NB]
        wsem = bufs_and_sems[3 * NB:4 * NB]
        wid = lax.axis_index("core") * 16 + lax.axis_index("subcore")
        row0 = wid * rows_per_tile

        pltpu.sync_copy(x_hbm.at[pl.ds(row0, rows_per_tile)], idx_v)

        def gather_start(bf, i):
            pltpu.make_async_copy(
                table_hbm.at[idx_v.at[i]], ibuf[bf], gsem[bf]).start()

        def gather_wait(bf, i):
            pltpu.make_async_copy(
                table_hbm.at[idx_v.at[i]], ibuf[bf], gsem[bf]).wait()

        def write_start(bf, i):
            pltpu.make_async_copy(
                obuf[bf], out_hbm.at[row0 + i], wsem[bf]).start()

        def write_wait(bf, i):
            pltpu.make_async_copy(
                obuf[bf], out_hbm.at[row0 + i], wsem[bf]).wait()

        for bf in range(NB):
            gather_start(bf, bf)

        for bf in range(NB):
            gather_wait(bf, bf)
            _scale_block(ibuf[bf], obuf[bf], s)
            gather_start(bf, bf + NB)
            write_start(bf, bf)

        @pl.loop(1, rounds)
        def _(r):
            i0 = r * NB
            for bf in range(NB):
                i = i0 + bf
                gather_wait(bf, i)
                write_wait(bf, i - NB)
                _scale_block(ibuf[bf], obuf[bf], s)

                @pl.when(i + NB < rows_per_tile)
                def _():
                    gather_start(bf, i + NB)

                write_start(bf, i)

        for bf in range(NB):
            write_wait(bf, rows_per_tile - NB + bf)

    return run(table, x.astype(jnp.int32))
